# Initial kernel scaffold; baseline (speedup 1.0000x reference)
import jax
import jax.numpy as jnp
from jax.experimental import pallas as pl


def kernel(x, edge_index, batch, embed_w, embed_b, bn1_g, bn1_b, W1, a1s, a1d, gb1, bn2_g, bn2_b, W2, a2s, a2d, gb2, k1, c1, l1w, l1b, k2, c2, l2w, l2b):
    raise NotImplementedError("write your pallas kernel here")


# speedup vs baseline: 16.3853x; 16.3853x over previous
import functools

import numpy as np
import jax
import jax.numpy as jnp
from jax import lax
from jax.experimental import pallas as pl
from jax.experimental.pallas import tpu as pltpu
from jax.experimental.pallas import tpu_sc as plsc

NN = 10000
NP = 10240
BLK = 512
NBLK = NP // BLK
NE = 320000
EP = 360448
ER = EP // 128
EPT = EP // 32
NCH = 11
CR = 8
CH = 1024
NG = 64
HID = 32
EPS = 1e-15


def _embed_body(x_ref, w_ref, b_ref, h_ref, s_ref):
    i = pl.program_id(0)
    h = jnp.dot(x_ref[...], w_ref[...], preferred_element_type=jnp.float32,
                precision=jax.lax.Precision.HIGHEST)
    h = h + b_ref[...]
    rows = i * BLK + lax.broadcasted_iota(jnp.int32, (BLK, 1), 0)
    h = jnp.where(rows < NN, h, 0.0)
    h_ref[...] = h
    part = jnp.concatenate(
        [jnp.sum(h, axis=0, keepdims=True), jnp.sum(h * h, axis=0, keepdims=True)], axis=0)

    @pl.when(i == 0)
    def _():
        s_ref[...] = part

    @pl.when(i > 0)
    def _():
        s_ref[...] = s_ref[...] + part


def _tc_embed(xp, w, b):
    return pl.pallas_call(
        _embed_body,
        grid=(NBLK,),
        in_specs=[
            pl.BlockSpec((BLK, 128), lambda i: (i, 0)),
            pl.BlockSpec((128, HID), lambda i: (0, 0)),
            pl.BlockSpec((1, HID), lambda i: (0, 0)),
        ],
        out_specs=[
            pl.BlockSpec((BLK, HID), lambda i: (i, 0)),
            pl.BlockSpec((2, HID), lambda i: (0, 0)),
        ],
        out_shape=[
            jax.ShapeDtypeStruct((NP, HID), jnp.float32),
            jax.ShapeDtypeStruct((2, HID), jnp.float32),
        ],
    )(xp, w, b)


def _var_body(h_ref, s_ref, v_ref):
    i = pl.program_id(0)
    mu = s_ref[...][0:1, :] * (1.0 / NN)
    dev = h_ref[...] - mu
    rows = i * BLK + lax.broadcasted_iota(jnp.int32, (BLK, 1), 0)
    dev2 = jnp.where(rows < NN, dev * dev, 0.0)
    part = jnp.sum(dev2, axis=0, keepdims=True)

    @pl.when(i == 0)
    def _():
        v_ref[...] = part

    @pl.when(i > 0)
    def _():
        v_ref[...] = v_ref[...] + part

    @pl.when(i == NBLK - 1)
    def _():
        v_ref[...] = v_ref[...] * (1.0 / NN)


def _tc_var(h, sums):
    return pl.pallas_call(
        _var_body,
        grid=(NBLK,),
        in_specs=[
            pl.BlockSpec((BLK, HID), lambda i: (i, 0)),
            pl.BlockSpec((2, HID), lambda i: (0, 0)),
        ],
        out_specs=pl.BlockSpec((1, HID), lambda i: (0, 0)),
        out_shape=jax.ShapeDtypeStruct((1, HID), jnp.float32),
    )(h, sums)


def _prep_body(h_ref, s_ref, v_ref, g_ref, b_ref, w_ref, as_ref, ad_ref,
               hw_ref, sal_ref, dal_ref):
    i = pl.program_id(0)
    s = s_ref[...]
    mu = s[0:1, :] * (1.0 / NN)
    var = v_ref[...]
    hn = (h_ref[...] - mu) / jnp.sqrt(var + 1e-5) * g_ref[...] + b_ref[...]
    hn = jnp.where(hn >= 0, hn, 0.01 * hn)
    hw = jnp.dot(hn, w_ref[...], preferred_element_type=jnp.float32,
                precision=jax.lax.Precision.HIGHEST)
    rows = i * BLK + lax.broadcasted_iota(jnp.int32, (BLK, 1), 0)
    hw = jnp.where(rows < NN, hw, 0.0)
    hw_ref[...] = hw
    sal_ref[...] = jnp.dot(hw, as_ref[...], preferred_element_type=jnp.float32,
                precision=jax.lax.Precision.HIGHEST)
    dal_ref[...] = jnp.dot(hw, ad_ref[...], preferred_element_type=jnp.float32,
                precision=jax.lax.Precision.HIGHEST)


def _tc_prep(h, sums, var, g, b, w, a_s, a_d):
    return pl.pallas_call(
        _prep_body,
        grid=(NBLK,),
        in_specs=[
            pl.BlockSpec((BLK, HID), lambda i: (i, 0)),
            pl.BlockSpec((2, HID), lambda i: (0, 0)),
            pl.BlockSpec((1, HID), lambda i: (0, 0)),
            pl.BlockSpec((1, HID), lambda i: (0, 0)),
            pl.BlockSpec((1, HID), lambda i: (0, 0)),
            pl.BlockSpec((HID, HID), lambda i: (0, 0)),
            pl.BlockSpec((HID, 1), lambda i: (0, 0)),
            pl.BlockSpec((HID, 1), lambda i: (0, 0)),
        ],
        out_specs=[
            pl.BlockSpec((BLK, HID), lambda i: (i, 0)),
            pl.BlockSpec((BLK, 1), lambda i: (i, 0)),
            pl.BlockSpec((BLK, 1), lambda i: (i, 0)),
        ],
        out_shape=[
            jax.ShapeDtypeStruct((NP, HID), jnp.float32),
            jax.ShapeDtypeStruct((NP, 1), jnp.float32),
            jax.ShapeDtypeStruct((NP, 1), jnp.float32),
        ],
    )(h, sums, var, g, b, w, a_s, a_d)


def _resid_body(h_ref, u0_ref, u1_ref, d0_ref, d1_ref, gb_ref, hn_ref, s_ref):
    i = pl.program_id(0)
    U = u0_ref[...][0, 0] + u1_ref[...][0, 0]
    den = d0_ref[...][0, 0, 0] + d1_ref[...][0, 0, 0]
    hnew = h_ref[...] + (U / (den[:, None] + 1e-16) + gb_ref[...])
    rows = i * BLK + lax.broadcasted_iota(jnp.int32, (BLK, 1), 0)
    hnew = jnp.where(rows < NN, hnew, 0.0)
    hn_ref[...] = hnew
    part = jnp.concatenate(
        [jnp.sum(hnew, axis=0, keepdims=True),
         jnp.sum(hnew * hnew, axis=0, keepdims=True)], axis=0)

    @pl.when(i == 0)
    def _():
        s_ref[...] = part

    @pl.when(i > 0)
    def _():
        s_ref[...] = s_ref[...] + part


def _tc_resid(h, u4, d4, gb):
    return pl.pallas_call(
        _resid_body,
        grid=(NBLK,),
        in_specs=[
            pl.BlockSpec((BLK, HID), lambda i: (i, 0)),
            pl.BlockSpec((1, 1, BLK, HID), lambda i: (0, i, 0, 0)),
            pl.BlockSpec((1, 1, BLK, HID), lambda i: (1, i, 0, 0)),
            pl.BlockSpec((1, 1, 1, BLK), lambda i: (0, i, 0, 0)),
            pl.BlockSpec((1, 1, 1, BLK), lambda i: (1, i, 0, 0)),
            pl.BlockSpec((1, HID), lambda i: (0, 0)),
        ],
        out_specs=[
            pl.BlockSpec((BLK, HID), lambda i: (i, 0)),
            pl.BlockSpec((2, HID), lambda i: (0, 0)),
        ],
        out_shape=[
            jax.ShapeDtypeStruct((NP, HID), jnp.float32),
            jax.ShapeDtypeStruct((2, HID), jnp.float32),
        ],
    )(h, u4, u4, d4, d4, gb)


def _pool_body(h_ref, u0_ref, u1_ref, d0_ref, d1_ref, gb_ref, bt_ref,
               kkT_ref, eh_ref, cc_ref, r_ref, q_ref, s_out_ref, pooled_ref):
    i = pl.program_id(0)
    U = u0_ref[...][0, 0] + u1_ref[...][0, 0]
    den = d0_ref[...][0, 0, 0] + d1_ref[...][0, 0, 0]
    h = h_ref[...] + (U / (den[:, None] + 1e-16) + gb_ref[...])
    rows = i * BLK + lax.broadcasted_iota(jnp.int32, (BLK, 1), 0)
    h = jnp.where(rows < NN, h, 0.0)
    kkT = kkT_ref[...]
    kk2 = jnp.sum(kkT * kkT, axis=0, keepdims=True)
    hh = jnp.sum(h * h, axis=1, keepdims=True)
    G = jnp.dot(h, kkT, preferred_element_type=jnp.float32,
                precision=jax.lax.Precision.HIGHEST)
    d2 = jnp.maximum(kk2 + hh - 2.0 * G, 0.0)
    dist = 1.0 / (1.0 + d2)
    dsum = jnp.dot(dist, eh_ref[...], preferred_element_type=jnp.float32,
                precision=jax.lax.Precision.HIGHEST)
    distn = dist / dsum
    Sl = jnp.dot(distn, cc_ref[...], preferred_element_type=jnp.float32,
                precision=jax.lax.Precision.HIGHEST)
    m = jnp.max(Sl, axis=1, keepdims=True)
    eS = jnp.exp(Sl - m)
    S = eS / jnp.sum(eS, axis=1, keepdims=True)
    S = jnp.where(rows < NN, S, 0.0)
    s_out_ref[...] = S
    bt = bt_ref[...][0]
    gid = lax.broadcasted_iota(jnp.int32, (NG, BLK), 0)
    Mt = jnp.where(gid == bt, 1.0, 0.0)
    T = jnp.dot(S, r_ref[...], preferred_element_type=jnp.float32,
                precision=jax.lax.Precision.HIGHEST) * \
        jnp.dot(h, q_ref[...], preferred_element_type=jnp.float32,
                precision=jax.lax.Precision.HIGHEST)
    part = jnp.dot(Mt, T, preferred_element_type=jnp.float32,
                precision=jax.lax.Precision.HIGHEST)

    @pl.when(i == 0)
    def _():
        pooled_ref[...] = part

    @pl.when(i > 0)
    def _():
        pooled_ref[...] = pooled_ref[...] + part


def _tc_pool(h, u4, d4, gb, bt, kkT, eh, cc, r, q):
    return pl.pallas_call(
        _pool_body,
        grid=(NBLK,),
        in_specs=[
            pl.BlockSpec((BLK, HID), lambda i: (i, 0)),
            pl.BlockSpec((1, 1, BLK, HID), lambda i: (0, i, 0, 0)),
            pl.BlockSpec((1, 1, BLK, HID), lambda i: (1, i, 0, 0)),
            pl.BlockSpec((1, 1, 1, BLK), lambda i: (0, i, 0, 0)),
            pl.BlockSpec((1, 1, 1, BLK), lambda i: (1, i, 0, 0)),
            pl.BlockSpec((1, HID), lambda i: (0, 0)),
            pl.BlockSpec((1, 1, BLK), lambda i: (i, 0, 0)),
            pl.BlockSpec((HID, 50), lambda i: (0, 0)),
            pl.BlockSpec((50, 50), lambda i: (0, 0)),
            pl.BlockSpec((50, 10), lambda i: (0, 0)),
            pl.BlockSpec((10, 320), lambda i: (0, 0)),
            pl.BlockSpec((HID, 320), lambda i: (0, 0)),
        ],
        out_specs=[
            pl.BlockSpec((BLK, 10), lambda i: (i, 0)),
            pl.BlockSpec((NG, 320), lambda i: (0, 0)),
        ],
        out_shape=[
            jax.ShapeDtypeStruct((NP, 10), jnp.float32),
            jax.ShapeDtypeStruct((NG, 320), jnp.float32),
        ],
    )(h, u4, u4, d4, d4, gb, bt, kkT, eh, cc, r, q)


def _head_body(p_ref, w1_ref, b1_ref, w2_ref, b2_ref, sm_ref, o_ref):
    x1 = jnp.dot(p_ref[...], w1_ref[...], preferred_element_type=jnp.float32,
                precision=jax.lax.Precision.HIGHEST) + b1_ref[...]
    x1 = jnp.where(x1 >= 0, x1, 0.01 * x1)
    z = jnp.dot(sm_ref[...], x1, preferred_element_type=jnp.float32,
                precision=jax.lax.Precision.HIGHEST)
    lg = jnp.dot(z, w2_ref[...], preferred_element_type=jnp.float32,
                precision=jax.lax.Precision.HIGHEST) + b2_ref[...]
    lg = jnp.where(lg >= 0, lg, 0.01 * lg)
    m = jnp.max(lg, axis=1, keepdims=True)
    sh = lg - m
    o_ref[...] = sh - jnp.log(jnp.sum(jnp.exp(sh), axis=1, keepdims=True))


def _tc_head(pooled2, w1, b1, w2, b2, sm):
    return pl.pallas_call(
        _head_body,
        grid=(1,),
        in_specs=[
            pl.BlockSpec((NG * 10, HID), lambda i: (0, 0)),
            pl.BlockSpec((HID, 80), lambda i: (0, 0)),
            pl.BlockSpec((1, 80), lambda i: (0, 0)),
            pl.BlockSpec((80, 10), lambda i: (0, 0)),
            pl.BlockSpec((1, 10), lambda i: (0, 0)),
            pl.BlockSpec((NG, NG * 10), lambda i: (0, 0)),
        ],
        out_specs=pl.BlockSpec((NG, 10), lambda i: (0, 0)),
        out_shape=jax.ShapeDtypeStruct((NG, 10), jnp.float32),
    )(pooled2, w1, b1, w2, b2, sm)


@functools.lru_cache(maxsize=1)
def _sc_gat_fn():
    mesh = plsc.VectorSubcoreMesh(core_axis_name="c", subcore_axis_name="s")

    @functools.partial(
        pl.kernel,
        out_type=(
            jax.ShapeDtypeStruct((2 * NP, HID), jnp.float32),
            jax.ShapeDtypeStruct((2 * NP,), jnp.float32),
        ),
        mesh=mesh,
        compiler_params=pltpu.CompilerParams(
            needs_layout_passes=False, use_tc_tiling_on_sc=False),
        scratch_types=[
            pltpu.VMEM((NP,), jnp.float32),
            pltpu.VMEM((NP,), jnp.float32),
            pltpu.VMEM((CR, 128), jnp.int32),
            pltpu.VMEM((CR, 128), jnp.int32),
            pltpu.VMEM((CH,), jnp.float32),
            pltpu.VMEM((CH, HID), jnp.float32),
            pltpu.VMEM_SHARED((NP, HID), jnp.float32),
            pltpu.VMEM_SHARED((NP,), jnp.float32),
            pltpu.SemaphoreType.DMA,
        ],
    )
    def _sc_gat(src_ref, dst_ref, sal_h, dal_h, hw_h, u_out, den_out,
                sal_v, dal_v, src_i, dst_i, e_v, rows_v, u_s, den_s, sem):
        c = lax.axis_index("c")
        s = lax.axis_index("s")
        wid = c * 16 + s
        nrows = NP // 16

        def zrow(j, carry):
            rows_v[j, 0:16] = jnp.zeros((16,), jnp.float32)
            rows_v[j, 16:32] = jnp.zeros((16,), jnp.float32)
            return carry

        lax.fori_loop(0, CH, zrow, 0)

        def zvec(v, carry):
            e_v[pl.ds(v * 16, 16)] = jnp.zeros((16,), jnp.float32)
            return carry

        lax.fori_loop(0, CH // 16, zvec, 0)
        pltpu.sync_copy(rows_v.at[pl.ds(0, nrows)], u_s.at[pl.ds(s * nrows, nrows)])
        pltpu.sync_copy(e_v.at[pl.ds(0, nrows)], den_s.at[pl.ds(s * nrows, nrows)])
        pltpu.sync_copy(sal_h, sal_v)
        pltpu.sync_copy(dal_h, dal_v)
        plsc.subcore_barrier()

        row_base = wid * (NCH * CR)

        def chunk(k, carry):
            r0 = row_base + k * CR
            pltpu.sync_copy(src_ref.at[pl.ds(r0, CR)], src_i)
            pltpu.sync_copy(dst_ref.at[pl.ds(r0, CR)], dst_i)
            for qq in range(CR):
                for tt in range(8):
                    sv = src_i[qq, pl.ds(tt * 16, 16)]
                    dv = dst_i[qq, pl.ds(tt * 16, 16)]
                    al = plsc.load_gather(sal_v, [sv]) + plsc.load_gather(dal_v, [dv])
                    al = jnp.where(al >= 0, al, 0.2 * al)
                    e_v[pl.ds(qq * 128 + tt * 16, 16)] = jnp.exp(al)
            for qq in range(CR):
                pltpu.sync_copy(e_v.at[pl.ds(qq * 128, 128)],
                                den_s.at[dst_i.at[qq]], add=True)
            for qq in range(CR):
                pltpu.async_copy(hw_h.at[src_i.at[qq]],
                                 rows_v.at[pl.ds(qq * 128, 128)], sem).wait()

            def scale(j, carry2):
                ev = plsc.load_gather(e_v, [jnp.full((16,), j, jnp.int32)])
                rows_v[j, 0:16] = rows_v[j, 0:16] * ev
                rows_v[j, 16:32] = rows_v[j, 16:32] * ev
                return carry2

            lax.fori_loop(0, CH, scale, 0)
            for qq in range(CR):
                pltpu.sync_copy(rows_v.at[pl.ds(qq * 128, 128)],
                                u_s.at[dst_i.at[qq]], add=True)
            return carry

        lax.fori_loop(0, NCH, chunk, 0)
        plsc.subcore_barrier()
        off = c * NP + s * nrows
        pltpu.sync_copy(u_s.at[pl.ds(s * nrows, nrows)], u_out.at[pl.ds(off, nrows)])
        pltpu.sync_copy(den_s.at[pl.ds(s * nrows, nrows)], den_out.at[pl.ds(off, nrows)])

    return _sc_gat


_EH = np.kron(np.eye(5, dtype=np.float32), np.ones((10, 10), np.float32))
_RK = np.kron(np.eye(10, dtype=np.float32), np.ones((1, HID), np.float32))
_QK = np.tile(np.eye(HID, dtype=np.float32), (1, 10))
_SM = np.kron(np.eye(NG, dtype=np.float32), np.ones((1, 10), np.float32))


def kernel(x, edge_index, batch, embed_w, embed_b, bn1_g, bn1_b, W1, a1s, a1d,
           gb1, bn2_g, bn2_b, W2, a2s, a2d, gb2, k1, c1, l1w, l1b, k2, c2,
           l2w, l2b):
    f32 = jnp.float32
    xp = jnp.zeros((NP, 128), f32).at[:NN].set(x)
    loops = jnp.arange(NN, dtype=jnp.int32)
    npad = EP - NE - NN
    srcr = jnp.concatenate(
        [edge_index[0], loops, jnp.full((npad,), NN, jnp.int32)]).reshape(ER, 128)
    dstr = jnp.concatenate(
        [edge_index[1], loops, jnp.full((npad,), NN, jnp.int32)]).reshape(ER, 128)
    btp = jnp.concatenate(
        [batch, jnp.full((NP - NN,), NG, jnp.int32)]).reshape(NBLK, 1, BLK)

    h0, sums0 = _tc_embed(xp, embed_w, embed_b.reshape(1, HID))
    var0 = _tc_var(h0, sums0)
    hw1, sal1, dal1 = _tc_prep(h0, sums0, var0, bn1_g.reshape(1, HID),
                               bn1_b.reshape(1, HID), W1,
                               a1s.reshape(HID, 1), a1d.reshape(HID, 1))
    u1, den1 = _sc_gat_fn()(srcr, dstr, sal1.reshape(NP), dal1.reshape(NP), hw1)
    h1, sums1 = _tc_resid(h0, u1.reshape(2, NBLK, BLK, HID),
                          den1.reshape(2, NBLK, 1, BLK), gb1.reshape(1, HID))
    var1 = _tc_var(h1, sums1)
    hw2, sal2, dal2 = _tc_prep(h1, sums1, var1, bn2_g.reshape(1, HID),
                               bn2_b.reshape(1, HID), W2,
                               a2s.reshape(HID, 1), a2d.reshape(HID, 1))
    u2, den2 = _sc_gat_fn()(srcr, dstr, sal2.reshape(NP), dal2.reshape(NP), hw2)

    kkT = k1.reshape(50, HID).T
    cc = jnp.repeat(c1, 10)[:, None] * jnp.asarray(np.tile(np.eye(10, dtype=np.float32), (5, 1)))
    S_p, pooled = _tc_pool(h1, u2.reshape(2, NBLK, BLK, HID),
                           den2.reshape(2, NBLK, 1, BLK), gb2.reshape(1, HID),
                           btp, kkT, jnp.asarray(_EH), cc,
                           jnp.asarray(_RK), jnp.asarray(_QK))
    out = _tc_head(pooled.reshape(NG * 10, HID), l1w, l1b.reshape(1, 80),
                   l2w, l2b.reshape(1, 10), jnp.asarray(_SM))

    S = S_p[:NN]
    counts = jnp.zeros((NG,), jnp.int32).at[batch].add(1)
    starts = jnp.concatenate([jnp.zeros((1,), jnp.int32), jnp.cumsum(counts)[:-1]])
    pos = (jnp.arange(NN, dtype=jnp.int32) - starts[batch]).astype(jnp.int32)
    S_dense = jnp.zeros((NG, NN, 10), f32).at[batch, pos].set(S)
    colsum = S_dense.sum(axis=1)
    csn = colsum[batch]
    P = S * S / jnp.maximum(csn, EPS)
    pd = P.sum(-1, keepdims=True)
    sS = S.sum(-1, keepdims=True)
    pd = jnp.where(sS == 0.0, 1.0, pd)
    P = P / pd
    Pc = jnp.clip(P, EPS, None)
    Sc = jnp.clip(S, EPS, None)
    loss = jnp.sum(Pc * (jnp.log(Pc) - jnp.log(Sc))) / NG
    return out, loss


# speedup vs baseline: 17.6323x; 1.0761x over previous
import functools

import numpy as np
import jax
import jax.numpy as jnp
from jax import lax
from jax.experimental import pallas as pl
from jax.experimental.pallas import tpu as pltpu
from jax.experimental.pallas import tpu_sc as plsc

NN = 10000
NP = 10240
BLK = 512
NBLK = NP // BLK
NE = 320000
EP = 360448
ER = EP // 128
EPT = EP // 32
NCH = 11
CR = 8
CH = 1024
NG = 64
HID = 32
EPS = 1e-15


def _embed_body(x_ref, w_ref, b_ref, h_ref, s_ref):
    i = pl.program_id(0)
    h = jnp.dot(x_ref[...], w_ref[...], preferred_element_type=jnp.float32,
                precision=jax.lax.Precision.HIGHEST)
    h = h + b_ref[...]
    rows = i * BLK + lax.broadcasted_iota(jnp.int32, (BLK, 1), 0)
    h = jnp.where(rows < NN, h, 0.0)
    h_ref[...] = h
    part = jnp.concatenate(
        [jnp.sum(h, axis=0, keepdims=True), jnp.sum(h * h, axis=0, keepdims=True)], axis=0)

    @pl.when(i == 0)
    def _():
        s_ref[...] = part

    @pl.when(i > 0)
    def _():
        s_ref[...] = s_ref[...] + part


def _tc_embed(xp, w, b):
    return pl.pallas_call(
        _embed_body,
        grid=(NBLK,),
        in_specs=[
            pl.BlockSpec((BLK, 128), lambda i: (i, 0)),
            pl.BlockSpec((128, HID), lambda i: (0, 0)),
            pl.BlockSpec((1, HID), lambda i: (0, 0)),
        ],
        out_specs=[
            pl.BlockSpec((BLK, HID), lambda i: (i, 0)),
            pl.BlockSpec((2, HID), lambda i: (0, 0)),
        ],
        out_shape=[
            jax.ShapeDtypeStruct((NP, HID), jnp.float32),
            jax.ShapeDtypeStruct((2, HID), jnp.float32),
        ],
    )(xp, w, b)


def _var_body(h_ref, s_ref, v_ref):
    i = pl.program_id(0)
    mu = s_ref[...][0:1, :] * (1.0 / NN)
    dev = h_ref[...] - mu
    rows = i * BLK + lax.broadcasted_iota(jnp.int32, (BLK, 1), 0)
    dev2 = jnp.where(rows < NN, dev * dev, 0.0)
    part = jnp.sum(dev2, axis=0, keepdims=True)

    @pl.when(i == 0)
    def _():
        v_ref[...] = part

    @pl.when(i > 0)
    def _():
        v_ref[...] = v_ref[...] + part

    @pl.when(i == NBLK - 1)
    def _():
        v_ref[...] = v_ref[...] * (1.0 / NN)


def _tc_var(h, sums):
    return pl.pallas_call(
        _var_body,
        grid=(NBLK,),
        in_specs=[
            pl.BlockSpec((BLK, HID), lambda i: (i, 0)),
            pl.BlockSpec((2, HID), lambda i: (0, 0)),
        ],
        out_specs=pl.BlockSpec((1, HID), lambda i: (0, 0)),
        out_shape=jax.ShapeDtypeStruct((1, HID), jnp.float32),
    )(h, sums)


def _prep_body(h_ref, s_ref, v_ref, g_ref, b_ref, w_ref, as_ref, ad_ref,
               hw_ref, sal_ref, dal_ref):
    i = pl.program_id(0)
    s = s_ref[...]
    mu = s[0:1, :] * (1.0 / NN)
    var = v_ref[...]
    hn = (h_ref[...] - mu) / jnp.sqrt(var + 1e-5) * g_ref[...] + b_ref[...]
    hn = jnp.where(hn >= 0, hn, 0.01 * hn)
    hw = jnp.dot(hn, w_ref[...], preferred_element_type=jnp.float32,
                precision=jax.lax.Precision.HIGHEST)
    rows = i * BLK + lax.broadcasted_iota(jnp.int32, (BLK, 1), 0)
    hw = jnp.where(rows < NN, hw, 0.0)
    hw_ref[...] = hw
    sal_ref[...] = jnp.dot(hw, as_ref[...], preferred_element_type=jnp.float32,
                precision=jax.lax.Precision.HIGHEST)
    dal_ref[...] = jnp.dot(hw, ad_ref[...], preferred_element_type=jnp.float32,
                precision=jax.lax.Precision.HIGHEST)


def _tc_prep(h, sums, var, g, b, w, a_s, a_d):
    return pl.pallas_call(
        _prep_body,
        grid=(NBLK,),
        in_specs=[
            pl.BlockSpec((BLK, HID), lambda i: (i, 0)),
            pl.BlockSpec((2, HID), lambda i: (0, 0)),
            pl.BlockSpec((1, HID), lambda i: (0, 0)),
            pl.BlockSpec((1, HID), lambda i: (0, 0)),
            pl.BlockSpec((1, HID), lambda i: (0, 0)),
            pl.BlockSpec((HID, HID), lambda i: (0, 0)),
            pl.BlockSpec((HID, 1), lambda i: (0, 0)),
            pl.BlockSpec((HID, 1), lambda i: (0, 0)),
        ],
        out_specs=[
            pl.BlockSpec((BLK, HID), lambda i: (i, 0)),
            pl.BlockSpec((BLK, 1), lambda i: (i, 0)),
            pl.BlockSpec((BLK, 1), lambda i: (i, 0)),
        ],
        out_shape=[
            jax.ShapeDtypeStruct((NP, HID), jnp.float32),
            jax.ShapeDtypeStruct((NP, 1), jnp.float32),
            jax.ShapeDtypeStruct((NP, 1), jnp.float32),
        ],
    )(h, sums, var, g, b, w, a_s, a_d)


def _resid_body(h_ref, u0_ref, u1_ref, d0_ref, d1_ref, gb_ref, hn_ref, s_ref):
    i = pl.program_id(0)
    U = u0_ref[...][0, 0] + u1_ref[...][0, 0]
    den = d0_ref[...][0, 0, 0] + d1_ref[...][0, 0, 0]
    hnew = h_ref[...] + (U / (den[:, None] + 1e-16) + gb_ref[...])
    rows = i * BLK + lax.broadcasted_iota(jnp.int32, (BLK, 1), 0)
    hnew = jnp.where(rows < NN, hnew, 0.0)
    hn_ref[...] = hnew
    part = jnp.concatenate(
        [jnp.sum(hnew, axis=0, keepdims=True),
         jnp.sum(hnew * hnew, axis=0, keepdims=True)], axis=0)

    @pl.when(i == 0)
    def _():
        s_ref[...] = part

    @pl.when(i > 0)
    def _():
        s_ref[...] = s_ref[...] + part


def _tc_resid(h, u4, d4, gb):
    return pl.pallas_call(
        _resid_body,
        grid=(NBLK,),
        in_specs=[
            pl.BlockSpec((BLK, HID), lambda i: (i, 0)),
            pl.BlockSpec((1, 1, BLK, HID), lambda i: (0, i, 0, 0)),
            pl.BlockSpec((1, 1, BLK, HID), lambda i: (1, i, 0, 0)),
            pl.BlockSpec((1, 1, 1, BLK), lambda i: (0, i, 0, 0)),
            pl.BlockSpec((1, 1, 1, BLK), lambda i: (1, i, 0, 0)),
            pl.BlockSpec((1, HID), lambda i: (0, 0)),
        ],
        out_specs=[
            pl.BlockSpec((BLK, HID), lambda i: (i, 0)),
            pl.BlockSpec((2, HID), lambda i: (0, 0)),
        ],
        out_shape=[
            jax.ShapeDtypeStruct((NP, HID), jnp.float32),
            jax.ShapeDtypeStruct((2, HID), jnp.float32),
        ],
    )(h, u4, u4, d4, d4, gb)


def _pool_body(h_ref, u0_ref, u1_ref, d0_ref, d1_ref, gb_ref, bt_ref,
               kkT_ref, eh_ref, cc_ref, r_ref, q_ref, s_out_ref, pooled_ref):
    i = pl.program_id(0)
    U = u0_ref[...][0, 0] + u1_ref[...][0, 0]
    den = d0_ref[...][0, 0, 0] + d1_ref[...][0, 0, 0]
    h = h_ref[...] + (U / (den[:, None] + 1e-16) + gb_ref[...])
    rows = i * BLK + lax.broadcasted_iota(jnp.int32, (BLK, 1), 0)
    h = jnp.where(rows < NN, h, 0.0)
    kkT = kkT_ref[...]
    kk2 = jnp.sum(kkT * kkT, axis=0, keepdims=True)
    hh = jnp.sum(h * h, axis=1, keepdims=True)
    G = jnp.dot(h, kkT, preferred_element_type=jnp.float32,
                precision=jax.lax.Precision.HIGHEST)
    d2 = jnp.maximum(kk2 + hh - 2.0 * G, 0.0)
    dist = 1.0 / (1.0 + d2)
    dsum = jnp.dot(dist, eh_ref[...], preferred_element_type=jnp.float32,
                precision=jax.lax.Precision.HIGHEST)
    distn = dist / dsum
    Sl = jnp.dot(distn, cc_ref[...], preferred_element_type=jnp.float32,
                precision=jax.lax.Precision.HIGHEST)
    m = jnp.max(Sl, axis=1, keepdims=True)
    eS = jnp.exp(Sl - m)
    S = eS / jnp.sum(eS, axis=1, keepdims=True)
    S = jnp.where(rows < NN, S, 0.0)
    s_out_ref[...] = S
    bt = bt_ref[...][0]
    gid = lax.broadcasted_iota(jnp.int32, (NG, BLK), 0)
    Mt = jnp.where(gid == bt, 1.0, 0.0)
    T = jnp.dot(S, r_ref[...], preferred_element_type=jnp.float32,
                precision=jax.lax.Precision.HIGHEST) * \
        jnp.dot(h, q_ref[...], preferred_element_type=jnp.float32,
                precision=jax.lax.Precision.HIGHEST)
    part = jnp.dot(Mt, T, preferred_element_type=jnp.float32,
                precision=jax.lax.Precision.HIGHEST)

    @pl.when(i == 0)
    def _():
        pooled_ref[...] = part

    @pl.when(i > 0)
    def _():
        pooled_ref[...] = pooled_ref[...] + part


def _tc_pool(h, u4, d4, gb, bt, kkT, eh, cc, r, q):
    return pl.pallas_call(
        _pool_body,
        grid=(NBLK,),
        in_specs=[
            pl.BlockSpec((BLK, HID), lambda i: (i, 0)),
            pl.BlockSpec((1, 1, BLK, HID), lambda i: (0, i, 0, 0)),
            pl.BlockSpec((1, 1, BLK, HID), lambda i: (1, i, 0, 0)),
            pl.BlockSpec((1, 1, 1, BLK), lambda i: (0, i, 0, 0)),
            pl.BlockSpec((1, 1, 1, BLK), lambda i: (1, i, 0, 0)),
            pl.BlockSpec((1, HID), lambda i: (0, 0)),
            pl.BlockSpec((1, 1, BLK), lambda i: (i, 0, 0)),
            pl.BlockSpec((HID, 50), lambda i: (0, 0)),
            pl.BlockSpec((50, 50), lambda i: (0, 0)),
            pl.BlockSpec((50, 10), lambda i: (0, 0)),
            pl.BlockSpec((10, 320), lambda i: (0, 0)),
            pl.BlockSpec((HID, 320), lambda i: (0, 0)),
        ],
        out_specs=[
            pl.BlockSpec((BLK, 10), lambda i: (i, 0)),
            pl.BlockSpec((NG, 320), lambda i: (0, 0)),
        ],
        out_shape=[
            jax.ShapeDtypeStruct((NP, 10), jnp.float32),
            jax.ShapeDtypeStruct((NG, 320), jnp.float32),
        ],
    )(h, u4, u4, d4, d4, gb, bt, kkT, eh, cc, r, q)


def _head_body(p_ref, w1_ref, b1_ref, w2_ref, b2_ref, sm_ref, o_ref):
    x1 = jnp.dot(p_ref[...], w1_ref[...], preferred_element_type=jnp.float32,
                precision=jax.lax.Precision.HIGHEST) + b1_ref[...]
    x1 = jnp.where(x1 >= 0, x1, 0.01 * x1)
    z = jnp.dot(sm_ref[...], x1, preferred_element_type=jnp.float32,
                precision=jax.lax.Precision.HIGHEST)
    lg = jnp.dot(z, w2_ref[...], preferred_element_type=jnp.float32,
                precision=jax.lax.Precision.HIGHEST) + b2_ref[...]
    lg = jnp.where(lg >= 0, lg, 0.01 * lg)
    m = jnp.max(lg, axis=1, keepdims=True)
    sh = lg - m
    o_ref[...] = sh - jnp.log(jnp.sum(jnp.exp(sh), axis=1, keepdims=True))


def _tc_head(pooled2, w1, b1, w2, b2, sm):
    return pl.pallas_call(
        _head_body,
        grid=(1,),
        in_specs=[
            pl.BlockSpec((NG * 10, HID), lambda i: (0, 0)),
            pl.BlockSpec((HID, 80), lambda i: (0, 0)),
            pl.BlockSpec((1, 80), lambda i: (0, 0)),
            pl.BlockSpec((80, 10), lambda i: (0, 0)),
            pl.BlockSpec((1, 10), lambda i: (0, 0)),
            pl.BlockSpec((NG, NG * 10), lambda i: (0, 0)),
        ],
        out_specs=pl.BlockSpec((NG, 10), lambda i: (0, 0)),
        out_shape=jax.ShapeDtypeStruct((NG, 10), jnp.float32),
    )(pooled2, w1, b1, w2, b2, sm)


@functools.lru_cache(maxsize=1)
def _sc_gat_fn():
    mesh = plsc.VectorSubcoreMesh(core_axis_name="c", subcore_axis_name="s")

    @functools.partial(
        pl.kernel,
        out_type=(
            jax.ShapeDtypeStruct((2 * NP, HID), jnp.float32),
            jax.ShapeDtypeStruct((2 * NP,), jnp.float32),
        ),
        mesh=mesh,
        compiler_params=pltpu.CompilerParams(
            needs_layout_passes=False, use_tc_tiling_on_sc=False),
        scratch_types=[
            pltpu.VMEM((NP,), jnp.float32),
            pltpu.VMEM((NP,), jnp.float32),
            pltpu.VMEM((CR, 128), jnp.int32),
            pltpu.VMEM((CR, 128), jnp.int32),
            pltpu.VMEM((CH,), jnp.float32),
            pltpu.VMEM((CH, HID), jnp.float32),
            pltpu.VMEM_SHARED((NP, HID), jnp.float32),
            pltpu.VMEM_SHARED((NP,), jnp.float32),
            pltpu.SemaphoreType.DMA,
        ],
    )
    def _sc_gat(src_ref, dst_ref, sal_h, dal_h, hw_h, u_out, den_out,
                sal_v, dal_v, src_i, dst_i, e_v, rows_v, u_s, den_s, sem):
        c = lax.axis_index("c")
        s = lax.axis_index("s")
        wid = c * 16 + s
        nrows = NP // 16

        def zrow(j, carry):
            rows_v[j, 0:16] = jnp.zeros((16,), jnp.float32)
            rows_v[j, 16:32] = jnp.zeros((16,), jnp.float32)
            return carry

        lax.fori_loop(0, CH, zrow, 0)

        def zvec(v, carry):
            e_v[pl.ds(v * 16, 16)] = jnp.zeros((16,), jnp.float32)
            return carry

        lax.fori_loop(0, CH // 16, zvec, 0)
        pltpu.sync_copy(rows_v.at[pl.ds(0, nrows)], u_s.at[pl.ds(s * nrows, nrows)])
        pltpu.sync_copy(e_v.at[pl.ds(0, nrows)], den_s.at[pl.ds(s * nrows, nrows)])
        pltpu.sync_copy(sal_h, sal_v)
        pltpu.sync_copy(dal_h, dal_v)
        plsc.subcore_barrier()

        row_base = wid * (NCH * CR)

        def chunk(k, carry):
            r0 = row_base + k * CR
            pltpu.sync_copy(src_ref.at[pl.ds(r0, CR)], src_i)
            pltpu.sync_copy(dst_ref.at[pl.ds(r0, CR)], dst_i)
            for qq in range(CR):
                for tt in range(8):
                    sv = src_i[qq, pl.ds(tt * 16, 16)]
                    dv = dst_i[qq, pl.ds(tt * 16, 16)]
                    al = plsc.load_gather(sal_v, [sv]) + plsc.load_gather(dal_v, [dv])
                    al = jnp.where(al >= 0, al, 0.2 * al)
                    e_v[pl.ds(qq * 128 + tt * 16, 16)] = jnp.exp(al)
            for qq in range(CR):
                pltpu.sync_copy(e_v.at[pl.ds(qq * 128, 128)],
                                den_s.at[dst_i.at[qq]], add=True)
            copies = [pltpu.async_copy(hw_h.at[src_i.at[qq]],
                                       rows_v.at[pl.ds(qq * 128, 128)], sem)
                      for qq in range(CR)]
            for cp in copies:
                cp.wait()

            @plsc.parallel_loop(0, CH, step=4, unroll=2)
            def _(j):
                for t in range(4):
                    jj = j + t
                    ev = plsc.load_gather(e_v, [jnp.full((16,), jj, jnp.int32)])
                    rows_v[jj, 0:16] = rows_v[jj, 0:16] * ev
                    rows_v[jj, 16:32] = rows_v[jj, 16:32] * ev
            for qq in range(CR):
                pltpu.sync_copy(rows_v.at[pl.ds(qq * 128, 128)],
                                u_s.at[dst_i.at[qq]], add=True)
            return carry

        lax.fori_loop(0, NCH, chunk, 0)
        plsc.subcore_barrier()
        off = c * NP + s * nrows
        pltpu.sync_copy(u_s.at[pl.ds(s * nrows, nrows)], u_out.at[pl.ds(off, nrows)])
        pltpu.sync_copy(den_s.at[pl.ds(s * nrows, nrows)], den_out.at[pl.ds(off, nrows)])

    return _sc_gat


_EH = np.kron(np.eye(5, dtype=np.float32), np.ones((10, 10), np.float32))
_RK = np.kron(np.eye(10, dtype=np.float32), np.ones((1, HID), np.float32))
_QK = np.tile(np.eye(HID, dtype=np.float32), (1, 10))
_SM = np.kron(np.eye(NG, dtype=np.float32), np.ones((1, 10), np.float32))


def kernel(x, edge_index, batch, embed_w, embed_b, bn1_g, bn1_b, W1, a1s, a1d,
           gb1, bn2_g, bn2_b, W2, a2s, a2d, gb2, k1, c1, l1w, l1b, k2, c2,
           l2w, l2b):
    f32 = jnp.float32
    xp = jnp.zeros((NP, 128), f32).at[:NN].set(x)
    loops = jnp.arange(NN, dtype=jnp.int32)
    npad = EP - NE - NN
    srcr = jnp.concatenate(
        [edge_index[0], loops, jnp.full((npad,), NN, jnp.int32)]).reshape(ER, 128)
    dstr = jnp.concatenate(
        [edge_index[1], loops, jnp.full((npad,), NN, jnp.int32)]).reshape(ER, 128)
    btp = jnp.concatenate(
        [batch, jnp.full((NP - NN,), NG, jnp.int32)]).reshape(NBLK, 1, BLK)

    h0, sums0 = _tc_embed(xp, embed_w, embed_b.reshape(1, HID))
    var0 = _tc_var(h0, sums0)
    hw1, sal1, dal1 = _tc_prep(h0, sums0, var0, bn1_g.reshape(1, HID),
                               bn1_b.reshape(1, HID), W1,
                               a1s.reshape(HID, 1), a1d.reshape(HID, 1))
    u1, den1 = _sc_gat_fn()(srcr, dstr, sal1.reshape(NP), dal1.reshape(NP), hw1)
    h1, sums1 = _tc_resid(h0, u1.reshape(2, NBLK, BLK, HID),
                          den1.reshape(2, NBLK, 1, BLK), gb1.reshape(1, HID))
    var1 = _tc_var(h1, sums1)
    hw2, sal2, dal2 = _tc_prep(h1, sums1, var1, bn2_g.reshape(1, HID),
                               bn2_b.reshape(1, HID), W2,
                               a2s.reshape(HID, 1), a2d.reshape(HID, 1))
    u2, den2 = _sc_gat_fn()(srcr, dstr, sal2.reshape(NP), dal2.reshape(NP), hw2)

    kkT = k1.reshape(50, HID).T
    cc = jnp.repeat(c1, 10)[:, None] * jnp.asarray(np.tile(np.eye(10, dtype=np.float32), (5, 1)))
    S_p, pooled = _tc_pool(h1, u2.reshape(2, NBLK, BLK, HID),
                           den2.reshape(2, NBLK, 1, BLK), gb2.reshape(1, HID),
                           btp, kkT, jnp.asarray(_EH), cc,
                           jnp.asarray(_RK), jnp.asarray(_QK))
    out = _tc_head(pooled.reshape(NG * 10, HID), l1w, l1b.reshape(1, 80),
                   l2w, l2b.reshape(1, 10), jnp.asarray(_SM))

    S = S_p[:NN]
    counts = jnp.zeros((NG,), jnp.int32).at[batch].add(1)
    starts = jnp.concatenate([jnp.zeros((1,), jnp.int32), jnp.cumsum(counts)[:-1]])
    pos = (jnp.arange(NN, dtype=jnp.int32) - starts[batch]).astype(jnp.int32)
    S_dense = jnp.zeros((NG, NN, 10), f32).at[batch, pos].set(S)
    colsum = S_dense.sum(axis=1)
    csn = colsum[batch]
    P = S * S / jnp.maximum(csn, EPS)
    pd = P.sum(-1, keepdims=True)
    sS = S.sum(-1, keepdims=True)
    pd = jnp.where(sS == 0.0, 1.0, pd)
    P = P / pd
    Pc = jnp.clip(P, EPS, None)
    Sc = jnp.clip(S, EPS, None)
    loss = jnp.sum(Pc * (jnp.log(Pc) - jnp.log(Sc))) / NG
    return out, loss


# speedup vs baseline: 17.6401x; 1.0004x over previous
import functools

import numpy as np
import jax
import jax.numpy as jnp
from jax import lax
from jax.experimental import pallas as pl
from jax.experimental.pallas import tpu as pltpu
from jax.experimental.pallas import tpu_sc as plsc

NN = 10000
NP = 10240
BLK = 512
NBLK = NP // BLK
NE = 320000
EP = 360448
ER = EP // 128
EPT = EP // 32
NCH = 11
CR = 8
CH = 1024
NG = 64
HID = 32
EPS = 1e-15


def _embed_body(x_ref, w_ref, b_ref, h_ref, s_ref):
    i = pl.program_id(0)
    h = jnp.dot(x_ref[...], w_ref[...], preferred_element_type=jnp.float32,
                precision=jax.lax.Precision.HIGHEST)
    h = h + b_ref[...]
    rows = i * BLK + lax.broadcasted_iota(jnp.int32, (BLK, 1), 0)
    h = jnp.where(rows < NN, h, 0.0)
    h_ref[...] = h
    part = jnp.concatenate(
        [jnp.sum(h, axis=0, keepdims=True), jnp.sum(h * h, axis=0, keepdims=True)], axis=0)

    @pl.when(i == 0)
    def _():
        s_ref[...] = part

    @pl.when(i > 0)
    def _():
        s_ref[...] = s_ref[...] + part


def _tc_embed(xp, w, b):
    return pl.pallas_call(
        _embed_body,
        grid=(NBLK,),
        in_specs=[
            pl.BlockSpec((BLK, 128), lambda i: (i, 0)),
            pl.BlockSpec((128, HID), lambda i: (0, 0)),
            pl.BlockSpec((1, HID), lambda i: (0, 0)),
        ],
        out_specs=[
            pl.BlockSpec((BLK, HID), lambda i: (i, 0)),
            pl.BlockSpec((2, HID), lambda i: (0, 0)),
        ],
        out_shape=[
            jax.ShapeDtypeStruct((NP, HID), jnp.float32),
            jax.ShapeDtypeStruct((2, HID), jnp.float32),
        ],
    )(xp, w, b)


def _var_body(h_ref, s_ref, v_ref):
    i = pl.program_id(0)
    mu = s_ref[...][0:1, :] * (1.0 / NN)
    dev = h_ref[...] - mu
    rows = i * BLK + lax.broadcasted_iota(jnp.int32, (BLK, 1), 0)
    dev2 = jnp.where(rows < NN, dev * dev, 0.0)
    part = jnp.sum(dev2, axis=0, keepdims=True)

    @pl.when(i == 0)
    def _():
        v_ref[...] = part

    @pl.when(i > 0)
    def _():
        v_ref[...] = v_ref[...] + part

    @pl.when(i == NBLK - 1)
    def _():
        v_ref[...] = v_ref[...] * (1.0 / NN)


def _tc_var(h, sums):
    return pl.pallas_call(
        _var_body,
        grid=(NBLK,),
        in_specs=[
            pl.BlockSpec((BLK, HID), lambda i: (i, 0)),
            pl.BlockSpec((2, HID), lambda i: (0, 0)),
        ],
        out_specs=pl.BlockSpec((1, HID), lambda i: (0, 0)),
        out_shape=jax.ShapeDtypeStruct((1, HID), jnp.float32),
    )(h, sums)


def _prep_body(h_ref, s_ref, v_ref, g_ref, b_ref, w_ref, as_ref, ad_ref,
               hw_ref, sal_ref, dal_ref):
    i = pl.program_id(0)
    s = s_ref[...]
    mu = s[0:1, :] * (1.0 / NN)
    var = v_ref[...]
    hn = (h_ref[...] - mu) / jnp.sqrt(var + 1e-5) * g_ref[...] + b_ref[...]
    hn = jnp.where(hn >= 0, hn, 0.01 * hn)
    hw = jnp.dot(hn, w_ref[...], preferred_element_type=jnp.float32,
                precision=jax.lax.Precision.HIGHEST)
    rows = i * BLK + lax.broadcasted_iota(jnp.int32, (BLK, 1), 0)
    hw = jnp.where(rows < NN, hw, 0.0)
    hw_ref[...] = hw
    sal_ref[...] = jnp.dot(hw, as_ref[...], preferred_element_type=jnp.float32,
                precision=jax.lax.Precision.HIGHEST)
    dal_ref[...] = jnp.dot(hw, ad_ref[...], preferred_element_type=jnp.float32,
                precision=jax.lax.Precision.HIGHEST)


def _tc_prep(h, sums, var, g, b, w, a_s, a_d):
    return pl.pallas_call(
        _prep_body,
        grid=(NBLK,),
        in_specs=[
            pl.BlockSpec((BLK, HID), lambda i: (i, 0)),
            pl.BlockSpec((2, HID), lambda i: (0, 0)),
            pl.BlockSpec((1, HID), lambda i: (0, 0)),
            pl.BlockSpec((1, HID), lambda i: (0, 0)),
            pl.BlockSpec((1, HID), lambda i: (0, 0)),
            pl.BlockSpec((HID, HID), lambda i: (0, 0)),
            pl.BlockSpec((HID, 1), lambda i: (0, 0)),
            pl.BlockSpec((HID, 1), lambda i: (0, 0)),
        ],
        out_specs=[
            pl.BlockSpec((BLK, HID), lambda i: (i, 0)),
            pl.BlockSpec((BLK, 1), lambda i: (i, 0)),
            pl.BlockSpec((BLK, 1), lambda i: (i, 0)),
        ],
        out_shape=[
            jax.ShapeDtypeStruct((NP, HID), jnp.float32),
            jax.ShapeDtypeStruct((NP, 1), jnp.float32),
            jax.ShapeDtypeStruct((NP, 1), jnp.float32),
        ],
    )(h, sums, var, g, b, w, a_s, a_d)


def _resid_body(h_ref, u0_ref, u1_ref, d0_ref, d1_ref, gb_ref, hn_ref, s_ref):
    i = pl.program_id(0)
    U = u0_ref[...][0, 0] + u1_ref[...][0, 0]
    den = d0_ref[...][0, 0, 0] + d1_ref[...][0, 0, 0]
    hnew = h_ref[...] + (U / (den[:, None] + 1e-16) + gb_ref[...])
    rows = i * BLK + lax.broadcasted_iota(jnp.int32, (BLK, 1), 0)
    hnew = jnp.where(rows < NN, hnew, 0.0)
    hn_ref[...] = hnew
    part = jnp.concatenate(
        [jnp.sum(hnew, axis=0, keepdims=True),
         jnp.sum(hnew * hnew, axis=0, keepdims=True)], axis=0)

    @pl.when(i == 0)
    def _():
        s_ref[...] = part

    @pl.when(i > 0)
    def _():
        s_ref[...] = s_ref[...] + part


def _tc_resid(h, u4, d4, gb):
    return pl.pallas_call(
        _resid_body,
        grid=(NBLK,),
        in_specs=[
            pl.BlockSpec((BLK, HID), lambda i: (i, 0)),
            pl.BlockSpec((1, 1, BLK, HID), lambda i: (0, i, 0, 0)),
            pl.BlockSpec((1, 1, BLK, HID), lambda i: (1, i, 0, 0)),
            pl.BlockSpec((1, 1, 1, BLK), lambda i: (0, i, 0, 0)),
            pl.BlockSpec((1, 1, 1, BLK), lambda i: (1, i, 0, 0)),
            pl.BlockSpec((1, HID), lambda i: (0, 0)),
        ],
        out_specs=[
            pl.BlockSpec((BLK, HID), lambda i: (i, 0)),
            pl.BlockSpec((2, HID), lambda i: (0, 0)),
        ],
        out_shape=[
            jax.ShapeDtypeStruct((NP, HID), jnp.float32),
            jax.ShapeDtypeStruct((2, HID), jnp.float32),
        ],
    )(h, u4, u4, d4, d4, gb)


def _pool_body(h_ref, u0_ref, u1_ref, d0_ref, d1_ref, gb_ref, bt_ref,
               kkT_ref, eh_ref, cc_ref, r_ref, q_ref, s_out_ref, pooled_ref):
    i = pl.program_id(0)
    U = u0_ref[...][0, 0] + u1_ref[...][0, 0]
    den = d0_ref[...][0, 0, 0] + d1_ref[...][0, 0, 0]
    h = h_ref[...] + (U / (den[:, None] + 1e-16) + gb_ref[...])
    rows = i * BLK + lax.broadcasted_iota(jnp.int32, (BLK, 1), 0)
    h = jnp.where(rows < NN, h, 0.0)
    kkT = kkT_ref[...]
    kk2 = jnp.sum(kkT * kkT, axis=0, keepdims=True)
    hh = jnp.sum(h * h, axis=1, keepdims=True)
    G = jnp.dot(h, kkT, preferred_element_type=jnp.float32,
                precision=jax.lax.Precision.HIGHEST)
    d2 = jnp.maximum(kk2 + hh - 2.0 * G, 0.0)
    dist = 1.0 / (1.0 + d2)
    dsum = jnp.dot(dist, eh_ref[...], preferred_element_type=jnp.float32,
                precision=jax.lax.Precision.HIGHEST)
    distn = dist / dsum
    Sl = jnp.dot(distn, cc_ref[...], preferred_element_type=jnp.float32,
                precision=jax.lax.Precision.HIGHEST)
    m = jnp.max(Sl, axis=1, keepdims=True)
    eS = jnp.exp(Sl - m)
    S = eS / jnp.sum(eS, axis=1, keepdims=True)
    S = jnp.where(rows < NN, S, 0.0)
    s_out_ref[...] = S
    bt = bt_ref[...][0]
    gid = lax.broadcasted_iota(jnp.int32, (NG, BLK), 0)
    Mt = jnp.where(gid == bt, 1.0, 0.0)
    T = jnp.dot(S, r_ref[...], preferred_element_type=jnp.float32,
                precision=jax.lax.Precision.HIGHEST) * \
        jnp.dot(h, q_ref[...], preferred_element_type=jnp.float32,
                precision=jax.lax.Precision.HIGHEST)
    part = jnp.dot(Mt, T, preferred_element_type=jnp.float32,
                precision=jax.lax.Precision.HIGHEST)

    @pl.when(i == 0)
    def _():
        pooled_ref[...] = part

    @pl.when(i > 0)
    def _():
        pooled_ref[...] = pooled_ref[...] + part


def _tc_pool(h, u4, d4, gb, bt, kkT, eh, cc, r, q):
    return pl.pallas_call(
        _pool_body,
        grid=(NBLK,),
        in_specs=[
            pl.BlockSpec((BLK, HID), lambda i: (i, 0)),
            pl.BlockSpec((1, 1, BLK, HID), lambda i: (0, i, 0, 0)),
            pl.BlockSpec((1, 1, BLK, HID), lambda i: (1, i, 0, 0)),
            pl.BlockSpec((1, 1, 1, BLK), lambda i: (0, i, 0, 0)),
            pl.BlockSpec((1, 1, 1, BLK), lambda i: (1, i, 0, 0)),
            pl.BlockSpec((1, HID), lambda i: (0, 0)),
            pl.BlockSpec((1, 1, BLK), lambda i: (i, 0, 0)),
            pl.BlockSpec((HID, 50), lambda i: (0, 0)),
            pl.BlockSpec((50, 50), lambda i: (0, 0)),
            pl.BlockSpec((50, 10), lambda i: (0, 0)),
            pl.BlockSpec((10, 320), lambda i: (0, 0)),
            pl.BlockSpec((HID, 320), lambda i: (0, 0)),
        ],
        out_specs=[
            pl.BlockSpec((BLK, 10), lambda i: (i, 0)),
            pl.BlockSpec((NG, 320), lambda i: (0, 0)),
        ],
        out_shape=[
            jax.ShapeDtypeStruct((NP, 10), jnp.float32),
            jax.ShapeDtypeStruct((NG, 320), jnp.float32),
        ],
    )(h, u4, u4, d4, d4, gb, bt, kkT, eh, cc, r, q)


def _head_body(p_ref, w1_ref, b1_ref, w2_ref, b2_ref, sm_ref, o_ref):
    x1 = jnp.dot(p_ref[...], w1_ref[...], preferred_element_type=jnp.float32,
                precision=jax.lax.Precision.HIGHEST) + b1_ref[...]
    x1 = jnp.where(x1 >= 0, x1, 0.01 * x1)
    z = jnp.dot(sm_ref[...], x1, preferred_element_type=jnp.float32,
                precision=jax.lax.Precision.HIGHEST)
    lg = jnp.dot(z, w2_ref[...], preferred_element_type=jnp.float32,
                precision=jax.lax.Precision.HIGHEST) + b2_ref[...]
    lg = jnp.where(lg >= 0, lg, 0.01 * lg)
    m = jnp.max(lg, axis=1, keepdims=True)
    sh = lg - m
    o_ref[...] = sh - jnp.log(jnp.sum(jnp.exp(sh), axis=1, keepdims=True))


def _tc_head(pooled2, w1, b1, w2, b2, sm):
    return pl.pallas_call(
        _head_body,
        grid=(1,),
        in_specs=[
            pl.BlockSpec((NG * 10, HID), lambda i: (0, 0)),
            pl.BlockSpec((HID, 80), lambda i: (0, 0)),
            pl.BlockSpec((1, 80), lambda i: (0, 0)),
            pl.BlockSpec((80, 10), lambda i: (0, 0)),
            pl.BlockSpec((1, 10), lambda i: (0, 0)),
            pl.BlockSpec((NG, NG * 10), lambda i: (0, 0)),
        ],
        out_specs=pl.BlockSpec((NG, 10), lambda i: (0, 0)),
        out_shape=jax.ShapeDtypeStruct((NG, 10), jnp.float32),
    )(pooled2, w1, b1, w2, b2, sm)


@functools.lru_cache(maxsize=1)
def _sc_gat_fn():
    mesh = plsc.VectorSubcoreMesh(core_axis_name="c", subcore_axis_name="s")

    @functools.partial(
        pl.kernel,
        out_type=(
            jax.ShapeDtypeStruct((2 * NP, HID), jnp.float32),
            jax.ShapeDtypeStruct((2 * NP,), jnp.float32),
        ),
        mesh=mesh,
        compiler_params=pltpu.CompilerParams(
            needs_layout_passes=False, use_tc_tiling_on_sc=False),
        scratch_types=[
            pltpu.VMEM((NP,), jnp.float32),
            pltpu.VMEM((NP,), jnp.float32),
            pltpu.VMEM((CR, 128), jnp.int32),
            pltpu.VMEM((CR, 128), jnp.int32),
            pltpu.VMEM((CH,), jnp.float32),
            pltpu.VMEM((CH, HID), jnp.float32),
            pltpu.VMEM_SHARED((NP, HID), jnp.float32),
            pltpu.VMEM_SHARED((NP,), jnp.float32),
            pltpu.SemaphoreType.DMA,
            pltpu.SemaphoreType.DMA,
        ],
    )
    def _sc_gat(src_ref, dst_ref, sal_h, dal_h, hw_h, u_out, den_out,
                sal_v, dal_v, src_i, dst_i, e_v, rows_v, u_s, den_s, sem, sem2):
        c = lax.axis_index("c")
        s = lax.axis_index("s")
        wid = c * 16 + s
        nrows = NP // 16

        def zrow(j, carry):
            rows_v[j, 0:16] = jnp.zeros((16,), jnp.float32)
            rows_v[j, 16:32] = jnp.zeros((16,), jnp.float32)
            return carry

        lax.fori_loop(0, CH, zrow, 0)

        def zvec(v, carry):
            e_v[pl.ds(v * 16, 16)] = jnp.zeros((16,), jnp.float32)
            return carry

        lax.fori_loop(0, CH // 16, zvec, 0)
        pltpu.sync_copy(rows_v.at[pl.ds(0, nrows)], u_s.at[pl.ds(s * nrows, nrows)])
        pltpu.sync_copy(e_v.at[pl.ds(0, nrows)], den_s.at[pl.ds(s * nrows, nrows)])
        pltpu.sync_copy(sal_h, sal_v)
        pltpu.sync_copy(dal_h, dal_v)
        plsc.subcore_barrier()

        row_base = wid * (NCH * CR)

        def chunk(k, carry):
            r0 = row_base + k * CR
            pltpu.sync_copy(src_ref.at[pl.ds(r0, CR)], src_i)
            pltpu.sync_copy(dst_ref.at[pl.ds(r0, CR)], dst_i)
            for qq in range(CR):
                for tt in range(8):
                    sv = src_i[qq, pl.ds(tt * 16, 16)]
                    dv = dst_i[qq, pl.ds(tt * 16, 16)]
                    al = plsc.load_gather(sal_v, [sv]) + plsc.load_gather(dal_v, [dv])
                    al = jnp.where(al >= 0, al, 0.2 * al)
                    e_v[pl.ds(qq * 128 + tt * 16, 16)] = jnp.exp(al)
            dscat = [pltpu.async_copy(e_v.at[pl.ds(qq * 128, 128)],
                                      den_s.at[dst_i.at[qq]], sem2, add=True)
                     for qq in range(CR)]
            copies = [pltpu.async_copy(hw_h.at[src_i.at[qq]],
                                       rows_v.at[pl.ds(qq * 128, 128)], sem)
                      for qq in range(CR)]
            for cp in copies:
                cp.wait()

            @plsc.parallel_loop(0, CH, step=4, unroll=2)
            def _(j):
                for t in range(4):
                    jj = j + t
                    ev = plsc.load_gather(e_v, [jnp.full((16,), jj, jnp.int32)])
                    rows_v[jj, 0:16] = rows_v[jj, 0:16] * ev
                    rows_v[jj, 16:32] = rows_v[jj, 16:32] * ev
            uscat = [pltpu.async_copy(rows_v.at[pl.ds(qq * 128, 128)],
                                      u_s.at[dst_i.at[qq]], sem2, add=True)
                     for qq in range(CR)]
            for cp in dscat:
                cp.wait()
            for cp in uscat:
                cp.wait()
            return carry

        lax.fori_loop(0, NCH, chunk, 0)
        plsc.subcore_barrier()
        off = c * NP + s * nrows
        pltpu.sync_copy(u_s.at[pl.ds(s * nrows, nrows)], u_out.at[pl.ds(off, nrows)])
        pltpu.sync_copy(den_s.at[pl.ds(s * nrows, nrows)], den_out.at[pl.ds(off, nrows)])

    return _sc_gat


_EH = np.kron(np.eye(5, dtype=np.float32), np.ones((10, 10), np.float32))
_RK = np.kron(np.eye(10, dtype=np.float32), np.ones((1, HID), np.float32))
_QK = np.tile(np.eye(HID, dtype=np.float32), (1, 10))
_SM = np.kron(np.eye(NG, dtype=np.float32), np.ones((1, 10), np.float32))


def kernel(x, edge_index, batch, embed_w, embed_b, bn1_g, bn1_b, W1, a1s, a1d,
           gb1, bn2_g, bn2_b, W2, a2s, a2d, gb2, k1, c1, l1w, l1b, k2, c2,
           l2w, l2b):
    f32 = jnp.float32
    xp = jnp.zeros((NP, 128), f32).at[:NN].set(x)
    loops = jnp.arange(NN, dtype=jnp.int32)
    npad = EP - NE - NN
    srcr = jnp.concatenate(
        [edge_index[0], loops, jnp.full((npad,), NN, jnp.int32)]).reshape(ER, 128)
    dstr = jnp.concatenate(
        [edge_index[1], loops, jnp.full((npad,), NN, jnp.int32)]).reshape(ER, 128)
    btp = jnp.concatenate(
        [batch, jnp.full((NP - NN,), NG, jnp.int32)]).reshape(NBLK, 1, BLK)

    h0, sums0 = _tc_embed(xp, embed_w, embed_b.reshape(1, HID))
    var0 = _tc_var(h0, sums0)
    hw1, sal1, dal1 = _tc_prep(h0, sums0, var0, bn1_g.reshape(1, HID),
                               bn1_b.reshape(1, HID), W1,
                               a1s.reshape(HID, 1), a1d.reshape(HID, 1))
    u1, den1 = _sc_gat_fn()(srcr, dstr, sal1.reshape(NP), dal1.reshape(NP), hw1)
    h1, sums1 = _tc_resid(h0, u1.reshape(2, NBLK, BLK, HID),
                          den1.reshape(2, NBLK, 1, BLK), gb1.reshape(1, HID))
    var1 = _tc_var(h1, sums1)
    hw2, sal2, dal2 = _tc_prep(h1, sums1, var1, bn2_g.reshape(1, HID),
                               bn2_b.reshape(1, HID), W2,
                               a2s.reshape(HID, 1), a2d.reshape(HID, 1))
    u2, den2 = _sc_gat_fn()(srcr, dstr, sal2.reshape(NP), dal2.reshape(NP), hw2)

    kkT = k1.reshape(50, HID).T
    cc = jnp.repeat(c1, 10)[:, None] * jnp.asarray(np.tile(np.eye(10, dtype=np.float32), (5, 1)))
    S_p, pooled = _tc_pool(h1, u2.reshape(2, NBLK, BLK, HID),
                           den2.reshape(2, NBLK, 1, BLK), gb2.reshape(1, HID),
                           btp, kkT, jnp.asarray(_EH), cc,
                           jnp.asarray(_RK), jnp.asarray(_QK))
    out = _tc_head(pooled.reshape(NG * 10, HID), l1w, l1b.reshape(1, 80),
                   l2w, l2b.reshape(1, 10), jnp.asarray(_SM))

    S = S_p[:NN]
    counts = jnp.zeros((NG,), jnp.int32).at[batch].add(1)
    starts = jnp.concatenate([jnp.zeros((1,), jnp.int32), jnp.cumsum(counts)[:-1]])
    pos = (jnp.arange(NN, dtype=jnp.int32) - starts[batch]).astype(jnp.int32)
    S_dense = jnp.zeros((NG, NN, 10), f32).at[batch, pos].set(S)
    colsum = S_dense.sum(axis=1)
    csn = colsum[batch]
    P = S * S / jnp.maximum(csn, EPS)
    pd = P.sum(-1, keepdims=True)
    sS = S.sum(-1, keepdims=True)
    pd = jnp.where(sS == 0.0, 1.0, pd)
    P = P / pd
    Pc = jnp.clip(P, EPS, None)
    Sc = jnp.clip(S, EPS, None)
    loss = jnp.sum(Pc * (jnp.log(Pc) - jnp.log(Sc))) / NG
    return out, loss
